# baseline (speedup 1.0000x reference)
import jax
import jax.numpy as jnp
from jax import lax
from jax.experimental import pallas as pl
from jax.experimental.pallas import tpu as pltpu
import jax.experimental.pallas.tpu_sc as plsc

NORMV = 100.0
NCORE = 2
NSUB = 16
NW = NCORE * NSUB
CH = 80
NPAD = 10240
EBLK = 2560
NBLK = 2000
L = 16
KB = 5


def _silu(z):
    return z * jax.nn.sigmoid(z)


def _sc_gather_pair(ta, tb, ia, ib):
    E = ia.shape[0]
    D = ta.shape[1]
    CHG = CH // 2
    per_w = E // NW
    iters = per_w // CHG
    assert per_w * NW == E and iters * CHG == per_w

    ia3 = ia.reshape(NW, iters, CHG)
    ib3 = ib.reshape(NW, iters, CHG)
    mesh = plsc.VectorSubcoreMesh(core_axis_name="c", subcore_axis_name="s")

    def body(ta_ref, tb_ref, ia_ref, ib_ref, oa_ref, ob_ref,
             iva, ivb, ra, rb, isem, ga, gb, wa, wb):
        wid = lax.axis_index("s") * NCORE + lax.axis_index("c")
        base = wid * per_w
        pltpu.async_copy(ia_ref.at[wid], iva, isem).wait()
        pltpu.async_copy(ib_ref.at[wid], ivb, isem).wait()

        def step(g, carry):
            i0 = g * KB
            dls = []
            for j in range(KB):
                dls.append((
                    pltpu.async_copy(ta_ref.at[iva.at[i0 + j]],
                                     ra.at[pl.ds(j * CHG, CHG)], ga),
                    pltpu.async_copy(tb_ref.at[ivb.at[i0 + j]],
                                     rb.at[pl.ds(j * CHG, CHG)], gb)))
            wrs = []
            for j in range(KB):
                da, db = dls[j]
                da.wait()
                db.wait()
                off = base + (i0 + j) * CHG
                wrs.append((
                    pltpu.async_copy(ra.at[pl.ds(j * CHG, CHG)],
                                     oa_ref.at[pl.ds(off, CHG)], wa),
                    pltpu.async_copy(rb.at[pl.ds(j * CHG, CHG)],
                                     ob_ref.at[pl.ds(off, CHG)], wb)))
            for da, db in wrs:
                da.wait()
                db.wait()
            return carry

        lax.fori_loop(0, iters // KB, step, 0)

    f = pl.kernel(
        body,
        out_type=(jax.ShapeDtypeStruct((E, D), ta.dtype),
                  jax.ShapeDtypeStruct((E, D), tb.dtype)),
        mesh=mesh,
        scratch_types=[
            pltpu.VMEM((iters, CHG), jnp.int32),
            pltpu.VMEM((iters, CHG), jnp.int32),
            pltpu.VMEM((KB * CHG, D), ta.dtype),
            pltpu.VMEM((KB * CHG, D), tb.dtype),
            pltpu.SemaphoreType.DMA,
            pltpu.SemaphoreType.DMA,
            pltpu.SemaphoreType.DMA,
            pltpu.SemaphoreType.DMA,
            pltpu.SemaphoreType.DMA,
        ],
        compiler_params=pltpu.CompilerParams(needs_layout_passes=False),
    )
    return f(ta, tb, ia3, ib3)


def _sc_scatter(vals, idx):
    E, D = vals.shape
    Dc = D // NCORE
    per_t = E // NSUB
    iters = per_t // CH
    rpt = NPAD // NSUB
    assert per_t * NSUB == E and iters * CH == per_t

    idx3 = idx.reshape(NSUB, iters, CH)
    mesh = plsc.VectorSubcoreMesh(core_axis_name="c", subcore_axis_name="s")

    def body(vals_ref, idx_ref, olo_ref, ohi_ref,
             iv, vh, acc, isem, ls, ss):
        c = lax.axis_index("c")
        s = lax.axis_index("s")
        base = s * per_t
        cb = c * Dc
        pltpu.async_copy(idx_ref.at[s], iv, isem).wait()
        zv = jnp.zeros((L,), jnp.float32)
        for j in range(2 * CH):
            for k in range(Dc // L):
                vh[j, pl.ds(k * L, L)] = zv
        zcs = []
        for k in range(rpt // (2 * CH)):
            zcs.append(pltpu.async_copy(
                vh.at[pl.ds(0, 2 * CH)],
                acc.at[pl.ds(s * rpt + k * 2 * CH, 2 * CH)], isem))
        for d in zcs:
            d.wait()
        plsc.subcore_barrier()

        def step(g, carry):
            i0 = g * KB
            dls = []
            for j in range(KB):
                off = base + (i0 + j) * CH
                dls.append(pltpu.async_copy(
                    vals_ref.at[pl.ds(off, CH), pl.ds(cb, Dc)],
                    vh.at[pl.ds(j * CH, CH)], ls))
            for j in range(KB):
                dls[j].wait()
                pltpu.async_copy(vh.at[pl.ds(j * CH, CH)],
                                 acc.at[iv.at[i0 + j]], ss, add=True).wait()
            return carry

        lax.fori_loop(0, iters // KB, step, 0)
        plsc.subcore_barrier()

        @pl.when(c == 0)
        def _():
            pltpu.async_copy(acc.at[pl.ds(s * rpt, rpt)],
                             olo_ref.at[pl.ds(s * rpt, rpt)], isem).wait()

        @pl.when(c == 1)
        def _():
            pltpu.async_copy(acc.at[pl.ds(s * rpt, rpt)],
                             ohi_ref.at[pl.ds(s * rpt, rpt)], isem).wait()

    f = pl.kernel(
        body,
        out_type=(jax.ShapeDtypeStruct((NPAD, Dc), jnp.float32),
                  jax.ShapeDtypeStruct((NPAD, Dc), jnp.float32)),
        mesh=mesh,
        scratch_types=[
            pltpu.VMEM((iters, CH), jnp.int32),
            pltpu.VMEM((KB * CH, Dc), jnp.float32),
            pltpu.VMEM_SHARED((NPAD, Dc), jnp.float32),
            pltpu.SemaphoreType.DMA,
            pltpu.SemaphoreType.DMA,
            pltpu.SemaphoreType.DMA,
        ],
        compiler_params=pltpu.CompilerParams(use_tc_tiling_on_sc=False,
                                             needs_layout_passes=False),
    )
    return f(vals, idx3)


def _sc_xdiff(xpad, ia, ib):
    N16 = xpad.shape[1]
    E = ia.shape[0]
    per_w = E // NW
    iters = per_w // CH
    assert per_w * NW == E and iters * CH == per_w

    ia3 = ia.reshape(NW, iters, CH)
    ib3 = ib.reshape(NW, iters, CH)
    mesh = plsc.VectorSubcoreMesh(core_axis_name="c", subcore_axis_name="s")

    def body(x_ref, ia_ref, ib_ref, o0_ref, o1_ref, o2_ref,
             iva, ivb, ra, rb, d0, d1, d2, isem, ga, gb):
        wid = lax.axis_index("s") * NCORE + lax.axis_index("c")
        base = wid * per_w
        pltpu.async_copy(ia_ref.at[wid], iva, isem).wait()
        pltpu.async_copy(ib_ref.at[wid], ivb, isem).wait()
        lanes = lax.iota(jnp.int32, L)
        dbufs = (d0, d1, d2)

        def step(g0, carry):
            i0 = g0 * KB
            dls = []
            for j in range(KB):
                dls.append((
                    pltpu.async_copy(x_ref.at[iva.at[i0 + j]],
                                     ra.at[pl.ds(j * CH, CH)], ga),
                    pltpu.async_copy(x_ref.at[ivb.at[i0 + j]],
                                     rb.at[pl.ds(j * CH, CH)], gb)))
            for j in range(KB):
                da, db = dls[j]
                da.wait()
                db.wait()
                for g in range(CH // L):
                    rows = j * CH + g * L + lanes
                    loc = (i0 + j) * CH + g * L
                    for k in range(3):
                        cols = jnp.full((L,), k, jnp.int32)
                        va = plsc.load_gather(ra, [rows, cols])
                        vb = plsc.load_gather(rb, [rows, cols])
                        dbufs[k][pl.ds(loc, L)] = va - vb
            return carry

        lax.fori_loop(0, iters // KB, step, 0)
        pltpu.async_copy(d0, o0_ref.at[pl.ds(base, per_w)], ga).wait()
        pltpu.async_copy(d1, o1_ref.at[pl.ds(base, per_w)], ga).wait()
        pltpu.async_copy(d2, o2_ref.at[pl.ds(base, per_w)], ga).wait()

    f = pl.kernel(
        body,
        out_type=(jax.ShapeDtypeStruct((E,), jnp.float32),) * 3,
        mesh=mesh,
        scratch_types=[
            pltpu.VMEM((iters, CH), jnp.int32),
            pltpu.VMEM((iters, CH), jnp.int32),
            pltpu.VMEM((KB * CH, N16), jnp.float32),
            pltpu.VMEM((KB * CH, N16), jnp.float32),
            pltpu.VMEM((per_w,), jnp.float32),
            pltpu.VMEM((per_w,), jnp.float32),
            pltpu.VMEM((per_w,), jnp.float32),
            pltpu.SemaphoreType.DMA,
            pltpu.SemaphoreType.DMA,
            pltpu.SemaphoreType.DMA,
        ],
        compiler_params=pltpu.CompilerParams(use_tc_tiling_on_sc=False,
                                             needs_layout_passes=False),
    )
    return f(xpad, ia3, ib3)


def _sc_xscatter(t0, t1, t2, idx):
    E = idx.shape[0]
    W = 16
    per_w = E // NW
    iters = per_w // CH
    rpt = NPAD // NSUB
    assert per_w * NW == E and iters * CH == per_w

    idx3 = idx.reshape(NW, iters, CH)
    mesh = plsc.VectorSubcoreMesh(core_axis_name="c", subcore_axis_name="s")

    def body(t0_ref, t1_ref, t2_ref, idx_ref, oa_ref, ob_ref,
             iv, tb0, tb1, tb2, vb, acc, isem, ss):
        c = lax.axis_index("c")
        s = lax.axis_index("s")
        wid = s * NCORE + c
        base = wid * per_w
        pltpu.async_copy(idx_ref.at[wid], iv, isem).wait()
        pltpu.async_copy(t0_ref.at[pl.ds(base, per_w)], tb0, isem).wait()
        pltpu.async_copy(t1_ref.at[pl.ds(base, per_w)], tb1, isem).wait()
        pltpu.async_copy(t2_ref.at[pl.ds(base, per_w)], tb2, isem).wait()
        lanes = lax.iota(jnp.int32, L)
        tbufs = (tb0, tb1, tb2)
        zv = jnp.zeros((L,), jnp.float32)
        for j in range(KB * CH):
            vb[j, :] = zv
        zcs = []
        for k in range(rpt // (2 * CH)):
            zcs.append(pltpu.async_copy(
                vb.at[pl.ds(0, 2 * CH)],
                acc.at[pl.ds(s * rpt + k * 2 * CH, 2 * CH)], isem))
        for d in zcs:
            d.wait()
        plsc.subcore_barrier()

        def step(g0, carry):
            i0 = g0 * KB
            for j in range(KB):
                for g in range(CH // L):
                    rows = j * CH + g * L + lanes
                    loc = (i0 + j) * CH + g * L
                    for k in range(3):
                        cols = jnp.full((L,), k, jnp.int32)
                        plsc.store_scatter(vb, [rows, cols],
                                           tbufs[k][pl.ds(loc, L)])
                pltpu.async_copy(vb.at[pl.ds(j * CH, CH)],
                                 acc.at[iv.at[i0 + j]], ss, add=True).wait()
            return carry

        lax.fori_loop(0, iters // KB, step, 0)
        plsc.subcore_barrier()

        @pl.when(c == 0)
        def _():
            pltpu.async_copy(acc.at[pl.ds(s * rpt, rpt)],
                             oa_ref.at[pl.ds(s * rpt, rpt)], isem).wait()

        @pl.when(c == 1)
        def _():
            pltpu.async_copy(acc.at[pl.ds(s * rpt, rpt)],
                             ob_ref.at[pl.ds(s * rpt, rpt)], isem).wait()

    f = pl.kernel(
        body,
        out_type=(jax.ShapeDtypeStruct((NPAD, W), jnp.float32),
                  jax.ShapeDtypeStruct((NPAD, W), jnp.float32)),
        mesh=mesh,
        scratch_types=[
            pltpu.VMEM((iters, CH), jnp.int32),
            pltpu.VMEM((per_w,), jnp.float32),
            pltpu.VMEM((per_w,), jnp.float32),
            pltpu.VMEM((per_w,), jnp.float32),
            pltpu.VMEM((KB * CH, W), jnp.float32),
            pltpu.VMEM_SHARED((NPAD, W), jnp.float32),
            pltpu.SemaphoreType.DMA,
            pltpu.SemaphoreType.DMA,
        ],
        compiler_params=pltpu.CompilerParams(use_tc_tiling_on_sc=False,
                                             needs_layout_passes=False),
    )
    return f(t0, t1, t2, idx3)


def _tc_pq(h, wa, wb, b1):
    N, H = h.shape

    def body(h_ref, wa_ref, wb_ref, b1_ref, p_ref, q_ref):
        hb = h_ref[...].astype(jnp.bfloat16)
        p_ref[...] = jnp.dot(hb, wa_ref[...].astype(jnp.bfloat16),
                             preferred_element_type=jnp.float32) + b1_ref[...]
        q_ref[...] = jnp.dot(hb, wb_ref[...].astype(jnp.bfloat16),
                             preferred_element_type=jnp.float32)

    return pl.pallas_call(
        body,
        grid=(N // NBLK,),
        in_specs=[pl.BlockSpec((NBLK, H), lambda i: (i, 0)),
                  pl.BlockSpec((H, H), lambda i: (0, 0)),
                  pl.BlockSpec((H, H), lambda i: (0, 0)),
                  pl.BlockSpec((1, H), lambda i: (0, 0))],
        out_specs=[pl.BlockSpec((NBLK, H), lambda i: (i, 0))] * 2,
        out_shape=[jax.ShapeDtypeStruct((N, H), jnp.float32)] * 2,
    )(h, wa, wb, b1)


def _edge_pre(gp_ref, gq_ref, d0_ref, d1_ref, d2_ref, ea_ref, wr_ref, we_ref):
    d0 = d0_ref[0]
    d1 = d1_ref[0]
    d2 = d2_ref[0]
    radial = d0 * d0 + d1 * d1 + d2 * d2
    rt = radial.T
    et = ea_ref[0].T
    wr = wr_ref[...]
    we = we_ref[...]
    gp = gp_ref[...]
    gq = gq_ref[...]
    parts = []
    for g in range(EBLK // 128):
        lo, hi = g * 128, (g + 1) * 128
        parts.append(gp[lo:hi, :] + gq[lo:hi, :]
                     + rt[:, g:g + 1] * wr + et[:, g:g + 1] * we)
    return jnp.concatenate(parts, axis=0), radial


def _tc_gcl_edge(gp, gq, d0, d1, d2, eaf, wr, we, w2, b2, awt, ab):
    E = gp.shape[0]
    H = w2.shape[0]
    GW = EBLK // 128

    def body(gp_ref, gq_ref, d0_ref, d1_ref, d2_ref, ea_ref, wr_ref, we_ref,
             w2_ref, b2_ref, aw_ref, ab_ref, ef_ref):
        pre, _ = _edge_pre(gp_ref, gq_ref, d0_ref, d1_ref, d2_ref, ea_ref,
                           wr_ref, we_ref)
        m1 = _silu(pre)
        mij = _silu(jnp.dot(m1.astype(jnp.bfloat16),
                            w2_ref[...].astype(jnp.bfloat16),
                            preferred_element_type=jnp.float32) + b2_ref[...])
        attp = jnp.sum(mij * aw_ref[...], axis=1, keepdims=True) + ab_ref[...]
        att = jax.nn.sigmoid(attp)
        ef_ref[...] = mij * att * (1.0 / NORMV)

    return pl.pallas_call(
        body,
        grid=(E // EBLK,),
        in_specs=[pl.BlockSpec((EBLK, H), lambda i: (i, 0)),
                  pl.BlockSpec((EBLK, H), lambda i: (i, 0)),
                  pl.BlockSpec((1, GW, 128), lambda i: (i, 0, 0)),
                  pl.BlockSpec((1, GW, 128), lambda i: (i, 0, 0)),
                  pl.BlockSpec((1, GW, 128), lambda i: (i, 0, 0)),
                  pl.BlockSpec((1, GW, 128), lambda i: (i, 0, 0)),
                  pl.BlockSpec((1, H), lambda i: (0, 0)),
                  pl.BlockSpec((1, H), lambda i: (0, 0)),
                  pl.BlockSpec((H, H), lambda i: (0, 0)),
                  pl.BlockSpec((1, H), lambda i: (0, 0)),
                  pl.BlockSpec((1, H), lambda i: (0, 0)),
                  pl.BlockSpec((1, 1), lambda i: (0, 0))],
        out_specs=pl.BlockSpec((EBLK, H), lambda i: (i, 0)),
        out_shape=jax.ShapeDtypeStruct((E, H), jnp.float32),
    )(gp, gq, d0, d1, d2, eaf, wr, we, w2, b2, awt, ab)


def _tc_node(h, alo, ahi, w1a, w1blo, w1bhi, nb1, nw2, nb2):
    N, H = h.shape
    Dc = alo.shape[1]

    def body(h_ref, alo_ref, ahi_ref, w1a_ref, w1blo_ref, w1bhi_ref, nb1_ref,
             nw2_ref, nb2_ref, out_ref):
        hb = h_ref[...]
        b16 = jnp.bfloat16
        t = _silu(jnp.dot(hb.astype(b16), w1a_ref[...].astype(b16),
                          preferred_element_type=jnp.float32)
                  + jnp.dot(alo_ref[...].astype(b16), w1blo_ref[...].astype(b16),
                            preferred_element_type=jnp.float32)
                  + jnp.dot(ahi_ref[...].astype(b16), w1bhi_ref[...].astype(b16),
                            preferred_element_type=jnp.float32)
                  + nb1_ref[...])
        out_ref[...] = hb + jnp.dot(
            t.astype(b16), nw2_ref[...].astype(b16),
            preferred_element_type=jnp.float32) + nb2_ref[...]

    return pl.pallas_call(
        body,
        grid=(N // NBLK,),
        in_specs=[pl.BlockSpec((NBLK, H), lambda i: (i, 0)),
                  pl.BlockSpec((NBLK, Dc), lambda i: (i, 0)),
                  pl.BlockSpec((NBLK, Dc), lambda i: (i, 0)),
                  pl.BlockSpec((H, H), lambda i: (0, 0)),
                  pl.BlockSpec((Dc, H), lambda i: (0, 0)),
                  pl.BlockSpec((Dc, H), lambda i: (0, 0)),
                  pl.BlockSpec((1, H), lambda i: (0, 0)),
                  pl.BlockSpec((H, H), lambda i: (0, 0)),
                  pl.BlockSpec((1, H), lambda i: (0, 0))],
        out_specs=pl.BlockSpec((NBLK, H), lambda i: (i, 0)),
        out_shape=jax.ShapeDtypeStruct((N, H), jnp.float32),
    )(h, alo, ahi, w1a, w1blo, w1bhi, nb1, nw2, nb2)


def _tc_eq_edge(gp, gq, d0, d1, d2, eaf, wr, we, w2, b2, w3t):
    E = gp.shape[0]
    H = w2.shape[0]
    GW = EBLK // 128

    def body(gp_ref, gq_ref, d0_ref, d1_ref, d2_ref, ea_ref, wr_ref, we_ref,
             w2_ref, b2_ref, w3_ref, t0_ref, t1_ref, t2_ref):
        pre, radial = _edge_pre(gp_ref, gq_ref, d0_ref, d1_ref, d2_ref,
                                ea_ref, wr_ref, we_ref)
        t1m = _silu(pre)
        t2m = _silu(jnp.dot(t1m.astype(jnp.bfloat16),
                            w2_ref[...].astype(jnp.bfloat16),
                            preferred_element_type=jnp.float32) + b2_ref[...])
        phi = jnp.sum(t2m * w3_ref[...], axis=1, keepdims=True)
        scale = 1.0 / ((jnp.sqrt(radial + 1e-8) + 1.0) * NORMV)
        phl = []
        for g in range(EBLK // 128):
            phl.append(phi[g * 128:(g + 1) * 128, :].T)
        phi_lane = jnp.concatenate(phl, axis=0) * scale
        t0_ref[0] = d0_ref[0] * phi_lane
        t1_ref[0] = d1_ref[0] * phi_lane
        t2_ref[0] = d2_ref[0] * phi_lane

    return pl.pallas_call(
        body,
        grid=(E // EBLK,),
        in_specs=[pl.BlockSpec((EBLK, H), lambda i: (i, 0)),
                  pl.BlockSpec((EBLK, H), lambda i: (i, 0)),
                  pl.BlockSpec((1, GW, 128), lambda i: (i, 0, 0)),
                  pl.BlockSpec((1, GW, 128), lambda i: (i, 0, 0)),
                  pl.BlockSpec((1, GW, 128), lambda i: (i, 0, 0)),
                  pl.BlockSpec((1, GW, 128), lambda i: (i, 0, 0)),
                  pl.BlockSpec((1, H), lambda i: (0, 0)),
                  pl.BlockSpec((1, H), lambda i: (0, 0)),
                  pl.BlockSpec((H, H), lambda i: (0, 0)),
                  pl.BlockSpec((1, H), lambda i: (0, 0)),
                  pl.BlockSpec((1, H), lambda i: (0, 0))],
        out_specs=[pl.BlockSpec((1, GW, 128), lambda i: (i, 0, 0))] * 3,
        out_shape=[jax.ShapeDtypeStruct((E // EBLK, GW, 128), jnp.float32)] * 3,
    )(gp, gq, d0, d1, d2, eaf, wr, we, w2, b2, w3t)


def _tc_xout(x, a0, a1):
    N = x.shape[0]

    def body(x_ref, a0_ref, a1_ref, out_ref):
        out_ref[...] = x_ref[...] + a0_ref[:, :3] + a1_ref[:, :3]

    return pl.pallas_call(
        body,
        grid=(N // NBLK,),
        in_specs=[pl.BlockSpec((NBLK, 3), lambda i: (i, 0)),
                  pl.BlockSpec((NBLK, 16), lambda i: (i, 0)),
                  pl.BlockSpec((NBLK, 16), lambda i: (i, 0))],
        out_specs=pl.BlockSpec((NBLK, 3), lambda i: (i, 0)),
        out_shape=jax.ShapeDtypeStruct((N, 3), jnp.float32),
    )(x, a0, a1)


def kernel(h, x, edge_index, edge_attr, params):
    N, H = h.shape
    E = edge_index.shape[1]
    row = edge_index[0]
    col = edge_index[1]
    eq = params['eq']

    xpad = jnp.pad(x, ((0, 0), (0, 13)))
    nb = E // EBLK
    gw = EBLK // 128
    d0f, d1f, d2f = _sc_xdiff(xpad, row, col)
    d0 = d0f.reshape(nb, gw, 128)
    d1 = d1f.reshape(nb, gw, 128)
    d2 = d2f.reshape(nb, gw, 128)
    eaf = edge_attr[:, 0].reshape(nb, gw, 128)

    for p in params['gcl']:
        ew1 = p['ew1']
        P, Q = _tc_pq(h, ew1[:H], ew1[H:2 * H], p['eb1'][None, :])
        gp, gq = _sc_gather_pair(P, Q, row, col)
        ef = _tc_gcl_edge(gp, gq, d0, d1, d2, eaf,
                          ew1[2 * H:2 * H + 1], ew1[2 * H + 1:],
                          p['ew2'], p['eb2'][None, :],
                          p['aw'].T, p['ab'][None, :])
        alo, ahi = _sc_scatter(ef, row)
        hc = H + H // NCORE
        h = _tc_node(h, alo, ahi, p['nw1'][:H], p['nw1'][H:hc],
                     p['nw1'][hc:], p['nb1'][None, :],
                     p['nw2'], p['nb2'][None, :])

    w1 = eq['w1']
    P, Q = _tc_pq(h, w1[:H], w1[H:2 * H], eq['b1'][None, :])
    gp, gq = _sc_gather_pair(P, Q, row, col)
    t0, t1, t2 = _tc_eq_edge(gp, gq, d0, d1, d2, eaf,
                             w1[2 * H:2 * H + 1], w1[2 * H + 1:],
                             eq['w2'], eq['b2'][None, :], eq['w3'].T)
    a0, a1 = _sc_xscatter(t0.reshape(E), t1.reshape(E), t2.reshape(E), row)
    xnew = _tc_xout(x, a0, a1)
    return h, xnew


# speedup vs baseline: 1.1055x; 1.1055x over previous
import jax
import jax.numpy as jnp
from jax import lax
from jax.experimental import pallas as pl
from jax.experimental.pallas import tpu as pltpu
import jax.experimental.pallas.tpu_sc as plsc

NORMV = 100.0
NCORE = 2
NSUB = 16
NW = NCORE * NSUB
CH = 80
NPAD = 10240
EBLK = 2560
NBLK = 2000
L = 16
KB = 5


def _silu(z):
    return z * jax.nn.sigmoid(z)


def _sc_gather_pair(ta, tb, ia, ib):
    E = ia.shape[0]
    D = ta.shape[1]
    CHG = CH // 2
    per_w = E // NW
    iters = per_w // CHG
    assert per_w * NW == E and iters * CHG == per_w

    ia3 = ia.reshape(NW, iters, CHG)
    ib3 = ib.reshape(NW, iters, CHG)
    mesh = plsc.VectorSubcoreMesh(core_axis_name="c", subcore_axis_name="s")

    def body(ta_ref, tb_ref, ia_ref, ib_ref, oa_ref, ob_ref,
             iva, ivb, ra, rb, isem, ga, gb, wa, wb):
        wid = lax.axis_index("s") * NCORE + lax.axis_index("c")
        base = wid * per_w
        pltpu.async_copy(ia_ref.at[wid], iva, isem).wait()
        pltpu.async_copy(ib_ref.at[wid], ivb, isem).wait()

        def step(g, carry):
            i0 = g * KB
            dls = []
            for j in range(KB):
                dls.append((
                    pltpu.async_copy(ta_ref.at[iva.at[i0 + j]],
                                     ra.at[pl.ds(j * CHG, CHG)], ga),
                    pltpu.async_copy(tb_ref.at[ivb.at[i0 + j]],
                                     rb.at[pl.ds(j * CHG, CHG)], gb)))
            wrs = []
            for j in range(KB):
                da, db = dls[j]
                da.wait()
                db.wait()
                off = base + (i0 + j) * CHG
                wrs.append((
                    pltpu.async_copy(ra.at[pl.ds(j * CHG, CHG)],
                                     oa_ref.at[pl.ds(off, CHG)], wa),
                    pltpu.async_copy(rb.at[pl.ds(j * CHG, CHG)],
                                     ob_ref.at[pl.ds(off, CHG)], wb)))
            for da, db in wrs:
                da.wait()
                db.wait()
            return carry

        lax.fori_loop(0, iters // KB, step, 0)

    f = pl.kernel(
        body,
        out_type=(jax.ShapeDtypeStruct((E, D), ta.dtype),
                  jax.ShapeDtypeStruct((E, D), tb.dtype)),
        mesh=mesh,
        scratch_types=[
            pltpu.VMEM((iters, CHG), jnp.int32),
            pltpu.VMEM((iters, CHG), jnp.int32),
            pltpu.VMEM((KB * CHG, D), ta.dtype),
            pltpu.VMEM((KB * CHG, D), tb.dtype),
            pltpu.SemaphoreType.DMA,
            pltpu.SemaphoreType.DMA,
            pltpu.SemaphoreType.DMA,
            pltpu.SemaphoreType.DMA,
            pltpu.SemaphoreType.DMA,
        ],
        compiler_params=pltpu.CompilerParams(needs_layout_passes=False),
    )
    return f(ta, tb, ia3, ib3)


def _sc_scatter(vals, idx):
    E, D = vals.shape
    Dc = D // NCORE
    per_t = E // NSUB
    iters = per_t // CH
    rpt = NPAD // NSUB
    assert per_t * NSUB == E and iters * CH == per_t

    idx3 = idx.reshape(NSUB, iters, CH)
    mesh = plsc.VectorSubcoreMesh(core_axis_name="c", subcore_axis_name="s")

    def body(vals_ref, idx_ref, olo_ref, ohi_ref,
             iv, vh, acc, isem, ls, ss):
        c = lax.axis_index("c")
        s = lax.axis_index("s")
        base = s * per_t
        cb = c * Dc
        pltpu.async_copy(idx_ref.at[s], iv, isem).wait()
        zv = jnp.zeros((L,), jnp.float32)
        for j in range(2 * CH):
            for k in range(Dc // L):
                vh[j, pl.ds(k * L, L)] = zv
        zcs = []
        for k in range(rpt // (2 * CH)):
            zcs.append(pltpu.async_copy(
                vh.at[pl.ds(0, 2 * CH)],
                acc.at[pl.ds(s * rpt + k * 2 * CH, 2 * CH)], isem))
        for d in zcs:
            d.wait()
        plsc.subcore_barrier()

        def step(g, carry):
            i0 = g * KB
            dls = []
            for j in range(KB):
                off = base + (i0 + j) * CH
                dls.append(pltpu.async_copy(
                    vals_ref.at[pl.ds(off, CH), pl.ds(cb, Dc)],
                    vh.at[pl.ds(j * CH, CH)], ls))
            for j in range(KB):
                dls[j].wait()
                pltpu.async_copy(vh.at[pl.ds(j * CH, CH)],
                                 acc.at[iv.at[i0 + j]], ss, add=True).wait()
            return carry

        lax.fori_loop(0, iters // KB, step, 0)
        plsc.subcore_barrier()

        @pl.when(c == 0)
        def _():
            pltpu.async_copy(acc.at[pl.ds(s * rpt, rpt)],
                             olo_ref.at[pl.ds(s * rpt, rpt)], isem).wait()

        @pl.when(c == 1)
        def _():
            pltpu.async_copy(acc.at[pl.ds(s * rpt, rpt)],
                             ohi_ref.at[pl.ds(s * rpt, rpt)], isem).wait()

    f = pl.kernel(
        body,
        out_type=(jax.ShapeDtypeStruct((NPAD, Dc), jnp.float32),
                  jax.ShapeDtypeStruct((NPAD, Dc), jnp.float32)),
        mesh=mesh,
        scratch_types=[
            pltpu.VMEM((iters, CH), jnp.int32),
            pltpu.VMEM((KB * CH, Dc), jnp.float32),
            pltpu.VMEM_SHARED((NPAD, Dc), jnp.float32),
            pltpu.SemaphoreType.DMA,
            pltpu.SemaphoreType.DMA,
            pltpu.SemaphoreType.DMA,
        ],
        compiler_params=pltpu.CompilerParams(use_tc_tiling_on_sc=False,
                                             needs_layout_passes=False),
    )
    return f(vals, idx3)


def _sc_xdiff(xpad, ia, ib):
    N16 = xpad.shape[1]
    E = ia.shape[0]
    per_w = E // NW
    iters = per_w // CH
    assert per_w * NW == E and iters * CH == per_w

    ia3 = ia.reshape(NW, iters, CH)
    ib3 = ib.reshape(NW, iters, CH)
    mesh = plsc.VectorSubcoreMesh(core_axis_name="c", subcore_axis_name="s")

    def body(x_ref, ia_ref, ib_ref, o0_ref, o1_ref, o2_ref,
             iva, ivb, ra, rb, d0, d1, d2, isem, ga, gb):
        wid = lax.axis_index("s") * NCORE + lax.axis_index("c")
        base = wid * per_w
        pltpu.async_copy(ia_ref.at[wid], iva, isem).wait()
        pltpu.async_copy(ib_ref.at[wid], ivb, isem).wait()
        lanes = lax.iota(jnp.int32, L)
        dbufs = (d0, d1, d2)

        def step(g0, carry):
            i0 = g0 * KB
            dls = []
            for j in range(KB):
                dls.append((
                    pltpu.async_copy(x_ref.at[iva.at[i0 + j]],
                                     ra.at[pl.ds(j * CH, CH)], ga),
                    pltpu.async_copy(x_ref.at[ivb.at[i0 + j]],
                                     rb.at[pl.ds(j * CH, CH)], gb)))
            for j in range(KB):
                da, db = dls[j]
                da.wait()
                db.wait()
                for g in range(CH // L):
                    rows = j * CH + g * L + lanes
                    loc = (i0 + j) * CH + g * L
                    for k in range(3):
                        cols = jnp.full((L,), k, jnp.int32)
                        va = plsc.load_gather(ra, [rows, cols])
                        vb = plsc.load_gather(rb, [rows, cols])
                        dbufs[k][pl.ds(loc, L)] = va - vb
            return carry

        lax.fori_loop(0, iters // KB, step, 0)
        pltpu.async_copy(d0, o0_ref.at[pl.ds(base, per_w)], ga).wait()
        pltpu.async_copy(d1, o1_ref.at[pl.ds(base, per_w)], ga).wait()
        pltpu.async_copy(d2, o2_ref.at[pl.ds(base, per_w)], ga).wait()

    f = pl.kernel(
        body,
        out_type=(jax.ShapeDtypeStruct((E,), jnp.float32),) * 3,
        mesh=mesh,
        scratch_types=[
            pltpu.VMEM((iters, CH), jnp.int32),
            pltpu.VMEM((iters, CH), jnp.int32),
            pltpu.VMEM((KB * CH, N16), jnp.float32),
            pltpu.VMEM((KB * CH, N16), jnp.float32),
            pltpu.VMEM((per_w,), jnp.float32),
            pltpu.VMEM((per_w,), jnp.float32),
            pltpu.VMEM((per_w,), jnp.float32),
            pltpu.SemaphoreType.DMA,
            pltpu.SemaphoreType.DMA,
            pltpu.SemaphoreType.DMA,
        ],
        compiler_params=pltpu.CompilerParams(use_tc_tiling_on_sc=False,
                                             needs_layout_passes=False),
    )
    return f(xpad, ia3, ib3)


def _sc_xscatter(t0, t1, t2, idx):
    E = idx.shape[0]
    W = 16
    per_w = E // NW
    iters = per_w // CH
    rpt = NPAD // NSUB
    assert per_w * NW == E and iters * CH == per_w

    idx3 = idx.reshape(NW, iters, CH)
    mesh = plsc.VectorSubcoreMesh(core_axis_name="c", subcore_axis_name="s")

    def body(t0_ref, t1_ref, t2_ref, idx_ref, oa_ref, ob_ref,
             iv, tb0, tb1, tb2, vb, acc, isem, ss):
        c = lax.axis_index("c")
        s = lax.axis_index("s")
        wid = s * NCORE + c
        base = wid * per_w
        pltpu.async_copy(idx_ref.at[wid], iv, isem).wait()
        pltpu.async_copy(t0_ref.at[pl.ds(base, per_w)], tb0, isem).wait()
        pltpu.async_copy(t1_ref.at[pl.ds(base, per_w)], tb1, isem).wait()
        pltpu.async_copy(t2_ref.at[pl.ds(base, per_w)], tb2, isem).wait()
        lanes = lax.iota(jnp.int32, L)
        tbufs = (tb0, tb1, tb2)
        zv = jnp.zeros((L,), jnp.float32)
        for j in range(KB * CH):
            vb[j, :] = zv
        zcs = []
        for k in range(rpt // (2 * CH)):
            zcs.append(pltpu.async_copy(
                vb.at[pl.ds(0, 2 * CH)],
                acc.at[pl.ds(s * rpt + k * 2 * CH, 2 * CH)], isem))
        for d in zcs:
            d.wait()
        plsc.subcore_barrier()

        def step(g0, carry):
            i0 = g0 * KB
            for j in range(KB):
                for g in range(CH // L):
                    rows = j * CH + g * L + lanes
                    loc = (i0 + j) * CH + g * L
                    for k in range(3):
                        cols = jnp.full((L,), k, jnp.int32)
                        plsc.store_scatter(vb, [rows, cols],
                                           tbufs[k][pl.ds(loc, L)])
                pltpu.async_copy(vb.at[pl.ds(j * CH, CH)],
                                 acc.at[iv.at[i0 + j]], ss, add=True).wait()
            return carry

        lax.fori_loop(0, iters // KB, step, 0)
        plsc.subcore_barrier()

        @pl.when(c == 0)
        def _():
            pltpu.async_copy(acc.at[pl.ds(s * rpt, rpt)],
                             oa_ref.at[pl.ds(s * rpt, rpt)], isem).wait()

        @pl.when(c == 1)
        def _():
            pltpu.async_copy(acc.at[pl.ds(s * rpt, rpt)],
                             ob_ref.at[pl.ds(s * rpt, rpt)], isem).wait()

    f = pl.kernel(
        body,
        out_type=(jax.ShapeDtypeStruct((NPAD, W), jnp.float32),
                  jax.ShapeDtypeStruct((NPAD, W), jnp.float32)),
        mesh=mesh,
        scratch_types=[
            pltpu.VMEM((iters, CH), jnp.int32),
            pltpu.VMEM((per_w,), jnp.float32),
            pltpu.VMEM((per_w,), jnp.float32),
            pltpu.VMEM((per_w,), jnp.float32),
            pltpu.VMEM((KB * CH, W), jnp.float32),
            pltpu.VMEM_SHARED((NPAD, W), jnp.float32),
            pltpu.SemaphoreType.DMA,
            pltpu.SemaphoreType.DMA,
        ],
        compiler_params=pltpu.CompilerParams(use_tc_tiling_on_sc=False,
                                             needs_layout_passes=False),
    )
    return f(t0, t1, t2, idx3)


def _tc_pq(h, wa, wb, b1):
    N, H = h.shape

    def body(h_ref, wa_ref, wb_ref, b1_ref, p_ref, q_ref):
        hb = h_ref[...].astype(jnp.bfloat16)
        p_ref[...] = jnp.dot(hb, wa_ref[...].astype(jnp.bfloat16),
                             preferred_element_type=jnp.float32) + b1_ref[...]
        q_ref[...] = jnp.dot(hb, wb_ref[...].astype(jnp.bfloat16),
                             preferred_element_type=jnp.float32)

    return pl.pallas_call(
        body,
        grid=(N // NBLK,),
        in_specs=[pl.BlockSpec((NBLK, H), lambda i: (i, 0)),
                  pl.BlockSpec((H, H), lambda i: (0, 0)),
                  pl.BlockSpec((H, H), lambda i: (0, 0)),
                  pl.BlockSpec((1, H), lambda i: (0, 0))],
        out_specs=[pl.BlockSpec((NBLK, H), lambda i: (i, 0))] * 2,
        out_shape=[jax.ShapeDtypeStruct((N, H), jnp.float32)] * 2,
    )(h, wa, wb, b1)


def _edge_pre(gp_ref, gq_ref, d0_ref, d1_ref, d2_ref, ea_ref, wr_ref, we_ref):
    d0 = d0_ref[0]
    d1 = d1_ref[0]
    d2 = d2_ref[0]
    radial = d0 * d0 + d1 * d1 + d2 * d2
    rt = radial.T
    et = ea_ref[0].T
    wr = wr_ref[...]
    we = we_ref[...]
    gp = gp_ref[...]
    gq = gq_ref[...]
    parts = []
    for g in range(EBLK // 128):
        lo, hi = g * 128, (g + 1) * 128
        parts.append(gp[lo:hi, :] + gq[lo:hi, :]
                     + rt[:, g:g + 1] * wr + et[:, g:g + 1] * we)
    return jnp.concatenate(parts, axis=0), radial


def _tc_gcl_edge(gp, gq, d0, d1, d2, eaf, wr, we, w2, b2, awt, ab):
    E = gp.shape[0]
    H = w2.shape[0]
    GW = EBLK // 128

    def body(gp_ref, gq_ref, d0_ref, d1_ref, d2_ref, ea_ref, wr_ref, we_ref,
             w2_ref, b2_ref, aw_ref, ab_ref, ef_ref):
        pre, _ = _edge_pre(gp_ref, gq_ref, d0_ref, d1_ref, d2_ref, ea_ref,
                           wr_ref, we_ref)
        m1 = _silu(pre)
        mij = _silu(jnp.dot(m1.astype(jnp.bfloat16),
                            w2_ref[...].astype(jnp.bfloat16),
                            preferred_element_type=jnp.float32) + b2_ref[...])
        attp = jnp.sum(mij * aw_ref[...], axis=1, keepdims=True) + ab_ref[...]
        att = jax.nn.sigmoid(attp)
        ef_ref[...] = mij * att * (1.0 / NORMV)

    return pl.pallas_call(
        body,
        grid=(E // EBLK,),
        in_specs=[pl.BlockSpec((EBLK, H), lambda i: (i, 0)),
                  pl.BlockSpec((EBLK, H), lambda i: (i, 0)),
                  pl.BlockSpec((1, GW, 128), lambda i: (i, 0, 0)),
                  pl.BlockSpec((1, GW, 128), lambda i: (i, 0, 0)),
                  pl.BlockSpec((1, GW, 128), lambda i: (i, 0, 0)),
                  pl.BlockSpec((1, GW, 128), lambda i: (i, 0, 0)),
                  pl.BlockSpec((1, H), lambda i: (0, 0)),
                  pl.BlockSpec((1, H), lambda i: (0, 0)),
                  pl.BlockSpec((H, H), lambda i: (0, 0)),
                  pl.BlockSpec((1, H), lambda i: (0, 0)),
                  pl.BlockSpec((1, H), lambda i: (0, 0)),
                  pl.BlockSpec((1, 1), lambda i: (0, 0))],
        out_specs=pl.BlockSpec((EBLK, H), lambda i: (i, 0)),
        out_shape=jax.ShapeDtypeStruct((E, H), jnp.float32),
    )(gp, gq, d0, d1, d2, eaf, wr, we, w2, b2, awt, ab)


def _tc_node(h, aggs, w1a, w1blo, w1bhi, nb1, nw2, nb2):
    N, H = h.shape
    Dc = aggs[0][0].shape[1]
    npairs = len(aggs)

    def body(h_ref, *refs):
        agg_refs = refs[:2 * npairs]
        (w1a_ref, w1blo_ref, w1bhi_ref, nb1_ref, nw2_ref, nb2_ref,
         out_ref) = refs[2 * npairs:]
        hb = h_ref[...]
        b16 = jnp.bfloat16
        alo = agg_refs[0][...]
        ahi = agg_refs[1][...]
        for j in range(1, npairs):
            alo = alo + agg_refs[2 * j][...]
            ahi = ahi + agg_refs[2 * j + 1][...]
        t = _silu(jnp.dot(hb.astype(b16), w1a_ref[...].astype(b16),
                          preferred_element_type=jnp.float32)
                  + jnp.dot(alo.astype(b16), w1blo_ref[...].astype(b16),
                            preferred_element_type=jnp.float32)
                  + jnp.dot(ahi.astype(b16), w1bhi_ref[...].astype(b16),
                            preferred_element_type=jnp.float32)
                  + nb1_ref[...])
        out_ref[...] = hb + jnp.dot(
            t.astype(b16), nw2_ref[...].astype(b16),
            preferred_element_type=jnp.float32) + nb2_ref[...]

    flat_aggs = [a for pair in aggs for a in pair]
    return pl.pallas_call(
        body,
        grid=(N // NBLK,),
        in_specs=[pl.BlockSpec((NBLK, H), lambda i: (i, 0))]
                 + [pl.BlockSpec((NBLK, Dc), lambda i: (i, 0))] * (2 * npairs)
                 + [pl.BlockSpec((H, H), lambda i: (0, 0)),
                    pl.BlockSpec((Dc, H), lambda i: (0, 0)),
                    pl.BlockSpec((Dc, H), lambda i: (0, 0)),
                    pl.BlockSpec((1, H), lambda i: (0, 0)),
                    pl.BlockSpec((H, H), lambda i: (0, 0)),
                    pl.BlockSpec((1, H), lambda i: (0, 0))],
        out_specs=pl.BlockSpec((NBLK, H), lambda i: (i, 0)),
        out_shape=jax.ShapeDtypeStruct((N, H), jnp.float32),
    )(h, *flat_aggs, w1a, w1blo, w1bhi, nb1, nw2, nb2)


def _tc_eq_edge(gp, gq, d0, d1, d2, eaf, wr, we, w2, b2, w3t):
    E = gp.shape[0]
    H = w2.shape[0]
    GW = EBLK // 128

    def body(gp_ref, gq_ref, d0_ref, d1_ref, d2_ref, ea_ref, wr_ref, we_ref,
             w2_ref, b2_ref, w3_ref, t0_ref, t1_ref, t2_ref):
        pre, radial = _edge_pre(gp_ref, gq_ref, d0_ref, d1_ref, d2_ref,
                                ea_ref, wr_ref, we_ref)
        t1m = _silu(pre)
        t2m = _silu(jnp.dot(t1m.astype(jnp.bfloat16),
                            w2_ref[...].astype(jnp.bfloat16),
                            preferred_element_type=jnp.float32) + b2_ref[...])
        phi = jnp.sum(t2m * w3_ref[...], axis=1, keepdims=True)
        scale = 1.0 / ((jnp.sqrt(radial + 1e-8) + 1.0) * NORMV)
        phl = []
        for g in range(EBLK // 128):
            phl.append(phi[g * 128:(g + 1) * 128, :].T)
        phi_lane = jnp.concatenate(phl, axis=0) * scale
        t0_ref[0] = d0_ref[0] * phi_lane
        t1_ref[0] = d1_ref[0] * phi_lane
        t2_ref[0] = d2_ref[0] * phi_lane

    return pl.pallas_call(
        body,
        grid=(E // EBLK,),
        in_specs=[pl.BlockSpec((EBLK, H), lambda i: (i, 0)),
                  pl.BlockSpec((EBLK, H), lambda i: (i, 0)),
                  pl.BlockSpec((1, GW, 128), lambda i: (i, 0, 0)),
                  pl.BlockSpec((1, GW, 128), lambda i: (i, 0, 0)),
                  pl.BlockSpec((1, GW, 128), lambda i: (i, 0, 0)),
                  pl.BlockSpec((1, GW, 128), lambda i: (i, 0, 0)),
                  pl.BlockSpec((1, H), lambda i: (0, 0)),
                  pl.BlockSpec((1, H), lambda i: (0, 0)),
                  pl.BlockSpec((H, H), lambda i: (0, 0)),
                  pl.BlockSpec((1, H), lambda i: (0, 0)),
                  pl.BlockSpec((1, H), lambda i: (0, 0))],
        out_specs=[pl.BlockSpec((1, GW, 128), lambda i: (i, 0, 0))] * 3,
        out_shape=[jax.ShapeDtypeStruct((E // EBLK, GW, 128), jnp.float32)] * 3,
    )(gp, gq, d0, d1, d2, eaf, wr, we, w2, b2, w3t)


def _tc_xout(x, parts):
    N = x.shape[0]

    def body(x_ref, *refs):
        out_ref = refs[-1]
        acc = x_ref[...]
        for r in refs[:-1]:
            acc = acc + r[:, :3]
        out_ref[...] = acc

    na = len(parts)
    return pl.pallas_call(
        body,
        grid=(N // NBLK,),
        in_specs=[pl.BlockSpec((NBLK, 3), lambda i: (i, 0))]
                 + [pl.BlockSpec((NBLK, 16), lambda i: (i, 0))] * na,
        out_specs=pl.BlockSpec((NBLK, 3), lambda i: (i, 0)),
        out_shape=jax.ShapeDtypeStruct((N, 3), jnp.float32),
    )(x, *parts)


def kernel(h, x, edge_index, edge_attr, params):
    N, H = h.shape
    E = edge_index.shape[1]
    row = edge_index[0]
    col = edge_index[1]
    eq = params['eq']

    xpad = jnp.pad(x, ((0, 0), (0, 13)))
    nb = E // EBLK
    gw = EBLK // 128
    d0f, d1f, d2f = _sc_xdiff(xpad, row, col)
    d0 = d0f.reshape(nb, gw, 128)
    d1 = d1f.reshape(nb, gw, 128)
    d2 = d2f.reshape(nb, gw, 128)
    eaf = edge_attr[:, 0].reshape(nb, gw, 128)

    E1 = (E * 3 // 5 // EBLK) * EBLK
    halves = ((0, E1), (E1, E))
    dsl = lambda a, lo, hi: a.reshape(E)[lo:hi].reshape(
        (hi - lo) // EBLK, EBLK // 128, 128)
    hc = H + H // NCORE

    for p in params['gcl']:
        ew1 = p['ew1']
        P, Q = _tc_pq(h, ew1[:H], ew1[H:2 * H], p['eb1'][None, :])
        gs = [_sc_gather_pair(P, Q, row[lo:hi], col[lo:hi])
              for lo, hi in halves]
        efs = [_tc_gcl_edge(gs[j][0], gs[j][1],
                            dsl(d0, *halves[j]), dsl(d1, *halves[j]),
                            dsl(d2, *halves[j]), dsl(eaf, *halves[j]),
                            ew1[2 * H:2 * H + 1], ew1[2 * H + 1:],
                            p['ew2'], p['eb2'][None, :],
                            p['aw'].T, p['ab'][None, :])
               for j in range(2)]
        aggs = [_sc_scatter(efs[j], row[halves[j][0]:halves[j][1]])
                for j in range(2)]
        h = _tc_node(h, aggs, p['nw1'][:H], p['nw1'][H:hc],
                     p['nw1'][hc:], p['nb1'][None, :],
                     p['nw2'], p['nb2'][None, :])

    w1 = eq['w1']
    P, Q = _tc_pq(h, w1[:H], w1[H:2 * H], eq['b1'][None, :])
    gs = [_sc_gather_pair(P, Q, row[lo:hi], col[lo:hi]) for lo, hi in halves]
    ts = [_tc_eq_edge(gs[j][0], gs[j][1],
                      dsl(d0, *halves[j]), dsl(d1, *halves[j]),
                      dsl(d2, *halves[j]), dsl(eaf, *halves[j]),
                      w1[2 * H:2 * H + 1], w1[2 * H + 1:],
                      eq['w2'], eq['b2'][None, :], eq['w3'].T)
          for j in range(2)]
    t0 = jnp.concatenate([ts[0][0].reshape(E1), ts[1][0].reshape(E - E1)])
    t1 = jnp.concatenate([ts[0][1].reshape(E1), ts[1][1].reshape(E - E1)])
    t2 = jnp.concatenate([ts[0][2].reshape(E1), ts[1][2].reshape(E - E1)])
    a0, a1 = _sc_xscatter(t0, t1, t2, row)
    xnew = _tc_xout(x, [a0, a1])
    return h, xnew


# speedup vs baseline: 1.1118x; 1.0057x over previous
import jax
import jax.numpy as jnp
from jax import lax
from jax.experimental import pallas as pl
from jax.experimental.pallas import tpu as pltpu
import jax.experimental.pallas.tpu_sc as plsc

NORMV = 100.0
NCORE = 2
NSUB = 16
NW = NCORE * NSUB
CH = 80
NPAD = 10240
EBLK = 2560
NBLK = 2000
L = 16
KB = 5
KBG = 10


def _silu(z):
    return z * jax.nn.sigmoid(z)


def _sc_gather_pair(ta, tb, ia, ib):
    E = ia.shape[0]
    D = ta.shape[1]
    CHG = CH // 2
    per_w = E // NW
    iters = per_w // CHG
    assert per_w * NW == E and iters * CHG == per_w

    ia3 = ia.reshape(NW, iters, CHG)
    ib3 = ib.reshape(NW, iters, CHG)
    mesh = plsc.VectorSubcoreMesh(core_axis_name="c", subcore_axis_name="s")

    def body(ta_ref, tb_ref, ia_ref, ib_ref, oa_ref, ob_ref,
             iva, ivb, ra, rb, isem, ga, gb, wa, wb):
        wid = lax.axis_index("s") * NCORE + lax.axis_index("c")
        base = wid * per_w
        pltpu.async_copy(ia_ref.at[wid], iva, isem).wait()
        pltpu.async_copy(ib_ref.at[wid], ivb, isem).wait()

        def step(g, carry):
            i0 = g * KB
            dls = []
            for j in range(KB):
                dls.append((
                    pltpu.async_copy(ta_ref.at[iva.at[i0 + j]],
                                     ra.at[pl.ds(j * CHG, CHG)], ga),
                    pltpu.async_copy(tb_ref.at[ivb.at[i0 + j]],
                                     rb.at[pl.ds(j * CHG, CHG)], gb)))
            wrs = []
            for j in range(KB):
                da, db = dls[j]
                da.wait()
                db.wait()
                off = base + (i0 + j) * CHG
                wrs.append((
                    pltpu.async_copy(ra.at[pl.ds(j * CHG, CHG)],
                                     oa_ref.at[pl.ds(off, CHG)], wa),
                    pltpu.async_copy(rb.at[pl.ds(j * CHG, CHG)],
                                     ob_ref.at[pl.ds(off, CHG)], wb)))
            for da, db in wrs:
                da.wait()
                db.wait()
            return carry

        lax.fori_loop(0, iters // KB, step, 0)

    f = pl.kernel(
        body,
        out_type=(jax.ShapeDtypeStruct((E, D), ta.dtype),
                  jax.ShapeDtypeStruct((E, D), tb.dtype)),
        mesh=mesh,
        scratch_types=[
            pltpu.VMEM((iters, CHG), jnp.int32),
            pltpu.VMEM((iters, CHG), jnp.int32),
            pltpu.VMEM((KB * CHG, D), ta.dtype),
            pltpu.VMEM((KB * CHG, D), tb.dtype),
            pltpu.SemaphoreType.DMA,
            pltpu.SemaphoreType.DMA,
            pltpu.SemaphoreType.DMA,
            pltpu.SemaphoreType.DMA,
            pltpu.SemaphoreType.DMA,
        ],
        compiler_params=pltpu.CompilerParams(needs_layout_passes=False),
    )
    return f(ta, tb, ia3, ib3)


def _sc_scatter(vals, idx):
    E, D = vals.shape
    Dc = D // NCORE
    per_t = E // NSUB
    iters = per_t // CH
    rpt = NPAD // NSUB
    assert per_t * NSUB == E and iters * CH == per_t

    idx3 = idx.reshape(NSUB, iters, CH)
    mesh = plsc.VectorSubcoreMesh(core_axis_name="c", subcore_axis_name="s")

    def body(vals_ref, idx_ref, olo_ref, ohi_ref,
             iv, vh, acc, isem, ls, ss):
        c = lax.axis_index("c")
        s = lax.axis_index("s")
        base = s * per_t
        cb = c * Dc
        pltpu.async_copy(idx_ref.at[s], iv, isem).wait()
        zv = jnp.zeros((L,), jnp.float32)
        for j in range(2 * CH):
            for k in range(Dc // L):
                vh[j, pl.ds(k * L, L)] = zv
        zcs = []
        for k in range(rpt // (2 * CH)):
            zcs.append(pltpu.async_copy(
                vh.at[pl.ds(0, 2 * CH)],
                acc.at[pl.ds(s * rpt + k * 2 * CH, 2 * CH)], isem))
        for d in zcs:
            d.wait()
        plsc.subcore_barrier()

        def step(g, carry):
            i0 = g * KBG
            dls = []
            for j in range(KBG):
                off = base + (i0 + j) * CH
                dls.append(pltpu.async_copy(
                    vals_ref.at[pl.ds(off, CH), pl.ds(cb, Dc)],
                    vh.at[pl.ds(j * CH, CH)], ls))
            for j in range(KBG):
                dls[j].wait()
                pltpu.async_copy(vh.at[pl.ds(j * CH, CH)],
                                 acc.at[iv.at[i0 + j]], ss, add=True).wait()
            return carry

        lax.fori_loop(0, iters // KBG, step, 0)
        plsc.subcore_barrier()

        @pl.when(c == 0)
        def _():
            pltpu.async_copy(acc.at[pl.ds(s * rpt, rpt)],
                             olo_ref.at[pl.ds(s * rpt, rpt)], isem).wait()

        @pl.when(c == 1)
        def _():
            pltpu.async_copy(acc.at[pl.ds(s * rpt, rpt)],
                             ohi_ref.at[pl.ds(s * rpt, rpt)], isem).wait()

    f = pl.kernel(
        body,
        out_type=(jax.ShapeDtypeStruct((NPAD, Dc), jnp.float32),
                  jax.ShapeDtypeStruct((NPAD, Dc), jnp.float32)),
        mesh=mesh,
        scratch_types=[
            pltpu.VMEM((iters, CH), jnp.int32),
            pltpu.VMEM((KBG * CH, Dc), jnp.float32),
            pltpu.VMEM_SHARED((NPAD, Dc), jnp.float32),
            pltpu.SemaphoreType.DMA,
            pltpu.SemaphoreType.DMA,
            pltpu.SemaphoreType.DMA,
        ],
        compiler_params=pltpu.CompilerParams(use_tc_tiling_on_sc=False,
                                             needs_layout_passes=False),
    )
    return f(vals, idx3)


def _sc_xdiff(xpad, ia, ib):
    N16 = xpad.shape[1]
    E = ia.shape[0]
    per_w = E // NW
    iters = per_w // CH
    assert per_w * NW == E and iters * CH == per_w

    ia3 = ia.reshape(NW, iters, CH)
    ib3 = ib.reshape(NW, iters, CH)
    mesh = plsc.VectorSubcoreMesh(core_axis_name="c", subcore_axis_name="s")

    def body(x_ref, ia_ref, ib_ref, o0_ref, o1_ref, o2_ref,
             iva, ivb, ra, rb, d0, d1, d2, isem, ga, gb):
        wid = lax.axis_index("s") * NCORE + lax.axis_index("c")
        base = wid * per_w
        pltpu.async_copy(ia_ref.at[wid], iva, isem).wait()
        pltpu.async_copy(ib_ref.at[wid], ivb, isem).wait()
        lanes = lax.iota(jnp.int32, L)
        dbufs = (d0, d1, d2)

        def step(g0, carry):
            i0 = g0 * KB
            dls = []
            for j in range(KB):
                dls.append((
                    pltpu.async_copy(x_ref.at[iva.at[i0 + j]],
                                     ra.at[pl.ds(j * CH, CH)], ga),
                    pltpu.async_copy(x_ref.at[ivb.at[i0 + j]],
                                     rb.at[pl.ds(j * CH, CH)], gb)))
            for j in range(KB):
                da, db = dls[j]
                da.wait()
                db.wait()
                for g in range(CH // L):
                    rows = j * CH + g * L + lanes
                    loc = (i0 + j) * CH + g * L
                    for k in range(3):
                        cols = jnp.full((L,), k, jnp.int32)
                        va = plsc.load_gather(ra, [rows, cols])
                        vb = plsc.load_gather(rb, [rows, cols])
                        dbufs[k][pl.ds(loc, L)] = va - vb
            return carry

        lax.fori_loop(0, iters // KB, step, 0)
        pltpu.async_copy(d0, o0_ref.at[pl.ds(base, per_w)], ga).wait()
        pltpu.async_copy(d1, o1_ref.at[pl.ds(base, per_w)], ga).wait()
        pltpu.async_copy(d2, o2_ref.at[pl.ds(base, per_w)], ga).wait()

    f = pl.kernel(
        body,
        out_type=(jax.ShapeDtypeStruct((E,), jnp.float32),) * 3,
        mesh=mesh,
        scratch_types=[
            pltpu.VMEM((iters, CH), jnp.int32),
            pltpu.VMEM((iters, CH), jnp.int32),
            pltpu.VMEM((KB * CH, N16), jnp.float32),
            pltpu.VMEM((KB * CH, N16), jnp.float32),
            pltpu.VMEM((per_w,), jnp.float32),
            pltpu.VMEM((per_w,), jnp.float32),
            pltpu.VMEM((per_w,), jnp.float32),
            pltpu.SemaphoreType.DMA,
            pltpu.SemaphoreType.DMA,
            pltpu.SemaphoreType.DMA,
        ],
        compiler_params=pltpu.CompilerParams(use_tc_tiling_on_sc=False,
                                             needs_layout_passes=False),
    )
    return f(xpad, ia3, ib3)


def _sc_xscatter(t0, t1, t2, idx):
    E = idx.shape[0]
    W = 16
    per_w = E // NW
    iters = per_w // CH
    rpt = NPAD // NSUB
    assert per_w * NW == E and iters * CH == per_w

    idx3 = idx.reshape(NW, iters, CH)
    mesh = plsc.VectorSubcoreMesh(core_axis_name="c", subcore_axis_name="s")

    def body(t0_ref, t1_ref, t2_ref, idx_ref, oa_ref, ob_ref,
             iv, tb0, tb1, tb2, vb, acc, isem, ss):
        c = lax.axis_index("c")
        s = lax.axis_index("s")
        wid = s * NCORE + c
        base = wid * per_w
        pltpu.async_copy(idx_ref.at[wid], iv, isem).wait()
        pltpu.async_copy(t0_ref.at[pl.ds(base, per_w)], tb0, isem).wait()
        pltpu.async_copy(t1_ref.at[pl.ds(base, per_w)], tb1, isem).wait()
        pltpu.async_copy(t2_ref.at[pl.ds(base, per_w)], tb2, isem).wait()
        lanes = lax.iota(jnp.int32, L)
        tbufs = (tb0, tb1, tb2)
        zv = jnp.zeros((L,), jnp.float32)
        for j in range(KB * CH):
            vb[j, :] = zv
        zcs = []
        for k in range(rpt // (2 * CH)):
            zcs.append(pltpu.async_copy(
                vb.at[pl.ds(0, 2 * CH)],
                acc.at[pl.ds(s * rpt + k * 2 * CH, 2 * CH)], isem))
        for d in zcs:
            d.wait()
        plsc.subcore_barrier()

        def step(g0, carry):
            i0 = g0 * KB
            for j in range(KB):
                for g in range(CH // L):
                    rows = j * CH + g * L + lanes
                    loc = (i0 + j) * CH + g * L
                    for k in range(3):
                        cols = jnp.full((L,), k, jnp.int32)
                        plsc.store_scatter(vb, [rows, cols],
                                           tbufs[k][pl.ds(loc, L)])
                pltpu.async_copy(vb.at[pl.ds(j * CH, CH)],
                                 acc.at[iv.at[i0 + j]], ss, add=True).wait()
            return carry

        lax.fori_loop(0, iters // KB, step, 0)
        plsc.subcore_barrier()

        @pl.when(c == 0)
        def _():
            pltpu.async_copy(acc.at[pl.ds(s * rpt, rpt)],
                             oa_ref.at[pl.ds(s * rpt, rpt)], isem).wait()

        @pl.when(c == 1)
        def _():
            pltpu.async_copy(acc.at[pl.ds(s * rpt, rpt)],
                             ob_ref.at[pl.ds(s * rpt, rpt)], isem).wait()

    f = pl.kernel(
        body,
        out_type=(jax.ShapeDtypeStruct((NPAD, W), jnp.float32),
                  jax.ShapeDtypeStruct((NPAD, W), jnp.float32)),
        mesh=mesh,
        scratch_types=[
            pltpu.VMEM((iters, CH), jnp.int32),
            pltpu.VMEM((per_w,), jnp.float32),
            pltpu.VMEM((per_w,), jnp.float32),
            pltpu.VMEM((per_w,), jnp.float32),
            pltpu.VMEM((KB * CH, W), jnp.float32),
            pltpu.VMEM_SHARED((NPAD, W), jnp.float32),
            pltpu.SemaphoreType.DMA,
            pltpu.SemaphoreType.DMA,
        ],
        compiler_params=pltpu.CompilerParams(use_tc_tiling_on_sc=False,
                                             needs_layout_passes=False),
    )
    return f(t0, t1, t2, idx3)


def _tc_pq(h, wa, wb, b1):
    N, H = h.shape

    def body(h_ref, wa_ref, wb_ref, b1_ref, p_ref, q_ref):
        hb = h_ref[...].astype(jnp.bfloat16)
        p_ref[...] = jnp.dot(hb, wa_ref[...].astype(jnp.bfloat16),
                             preferred_element_type=jnp.float32) + b1_ref[...]
        q_ref[...] = jnp.dot(hb, wb_ref[...].astype(jnp.bfloat16),
                             preferred_element_type=jnp.float32)

    return pl.pallas_call(
        body,
        grid=(N // NBLK,),
        in_specs=[pl.BlockSpec((NBLK, H), lambda i: (i, 0)),
                  pl.BlockSpec((H, H), lambda i: (0, 0)),
                  pl.BlockSpec((H, H), lambda i: (0, 0)),
                  pl.BlockSpec((1, H), lambda i: (0, 0))],
        out_specs=[pl.BlockSpec((NBLK, H), lambda i: (i, 0))] * 2,
        out_shape=[jax.ShapeDtypeStruct((N, H), jnp.float32)] * 2,
    )(h, wa, wb, b1)


def _edge_pre(gp_ref, gq_ref, d0_ref, d1_ref, d2_ref, ea_ref, wr_ref, we_ref):
    d0 = d0_ref[0]
    d1 = d1_ref[0]
    d2 = d2_ref[0]
    radial = d0 * d0 + d1 * d1 + d2 * d2
    rt = radial.T
    et = ea_ref[0].T
    wr = wr_ref[...]
    we = we_ref[...]
    gp = gp_ref[...]
    gq = gq_ref[...]
    parts = []
    for g in range(EBLK // 128):
        lo, hi = g * 128, (g + 1) * 128
        parts.append(gp[lo:hi, :] + gq[lo:hi, :]
                     + rt[:, g:g + 1] * wr + et[:, g:g + 1] * we)
    return jnp.concatenate(parts, axis=0), radial


def _tc_gcl_edge(gp, gq, d0, d1, d2, eaf, wr, we, w2, b2, awt, ab):
    E = gp.shape[0]
    H = w2.shape[0]
    GW = EBLK // 128

    def body(gp_ref, gq_ref, d0_ref, d1_ref, d2_ref, ea_ref, wr_ref, we_ref,
             w2_ref, b2_ref, aw_ref, ab_ref, ef_ref):
        pre, _ = _edge_pre(gp_ref, gq_ref, d0_ref, d1_ref, d2_ref, ea_ref,
                           wr_ref, we_ref)
        m1 = _silu(pre)
        mij = _silu(jnp.dot(m1.astype(jnp.bfloat16),
                            w2_ref[...].astype(jnp.bfloat16),
                            preferred_element_type=jnp.float32) + b2_ref[...])
        attp = jnp.sum(mij * aw_ref[...], axis=1, keepdims=True) + ab_ref[...]
        att = jax.nn.sigmoid(attp)
        ef_ref[...] = mij * att * (1.0 / NORMV)

    return pl.pallas_call(
        body,
        grid=(E // EBLK,),
        in_specs=[pl.BlockSpec((EBLK, H), lambda i: (i, 0)),
                  pl.BlockSpec((EBLK, H), lambda i: (i, 0)),
                  pl.BlockSpec((1, GW, 128), lambda i: (i, 0, 0)),
                  pl.BlockSpec((1, GW, 128), lambda i: (i, 0, 0)),
                  pl.BlockSpec((1, GW, 128), lambda i: (i, 0, 0)),
                  pl.BlockSpec((1, GW, 128), lambda i: (i, 0, 0)),
                  pl.BlockSpec((1, H), lambda i: (0, 0)),
                  pl.BlockSpec((1, H), lambda i: (0, 0)),
                  pl.BlockSpec((H, H), lambda i: (0, 0)),
                  pl.BlockSpec((1, H), lambda i: (0, 0)),
                  pl.BlockSpec((1, H), lambda i: (0, 0)),
                  pl.BlockSpec((1, 1), lambda i: (0, 0))],
        out_specs=pl.BlockSpec((EBLK, H), lambda i: (i, 0)),
        out_shape=jax.ShapeDtypeStruct((E, H), jnp.float32),
    )(gp, gq, d0, d1, d2, eaf, wr, we, w2, b2, awt, ab)


def _tc_node(h, aggs, w1a, w1blo, w1bhi, nb1, nw2, nb2):
    N, H = h.shape
    Dc = aggs[0][0].shape[1]
    npairs = len(aggs)

    def body(h_ref, *refs):
        agg_refs = refs[:2 * npairs]
        (w1a_ref, w1blo_ref, w1bhi_ref, nb1_ref, nw2_ref, nb2_ref,
         out_ref) = refs[2 * npairs:]
        hb = h_ref[...]
        b16 = jnp.bfloat16
        alo = agg_refs[0][...]
        ahi = agg_refs[1][...]
        for j in range(1, npairs):
            alo = alo + agg_refs[2 * j][...]
            ahi = ahi + agg_refs[2 * j + 1][...]
        t = _silu(jnp.dot(hb.astype(b16), w1a_ref[...].astype(b16),
                          preferred_element_type=jnp.float32)
                  + jnp.dot(alo.astype(b16), w1blo_ref[...].astype(b16),
                            preferred_element_type=jnp.float32)
                  + jnp.dot(ahi.astype(b16), w1bhi_ref[...].astype(b16),
                            preferred_element_type=jnp.float32)
                  + nb1_ref[...])
        out_ref[...] = hb + jnp.dot(
            t.astype(b16), nw2_ref[...].astype(b16),
            preferred_element_type=jnp.float32) + nb2_ref[...]

    flat_aggs = [a for pair in aggs for a in pair]
    return pl.pallas_call(
        body,
        grid=(N // NBLK,),
        in_specs=[pl.BlockSpec((NBLK, H), lambda i: (i, 0))]
                 + [pl.BlockSpec((NBLK, Dc), lambda i: (i, 0))] * (2 * npairs)
                 + [pl.BlockSpec((H, H), lambda i: (0, 0)),
                    pl.BlockSpec((Dc, H), lambda i: (0, 0)),
                    pl.BlockSpec((Dc, H), lambda i: (0, 0)),
                    pl.BlockSpec((1, H), lambda i: (0, 0)),
                    pl.BlockSpec((H, H), lambda i: (0, 0)),
                    pl.BlockSpec((1, H), lambda i: (0, 0))],
        out_specs=pl.BlockSpec((NBLK, H), lambda i: (i, 0)),
        out_shape=jax.ShapeDtypeStruct((N, H), jnp.float32),
    )(h, *flat_aggs, w1a, w1blo, w1bhi, nb1, nw2, nb2)


def _tc_eq_edge(gp, gq, d0, d1, d2, eaf, wr, we, w2, b2, w3t):
    E = gp.shape[0]
    H = w2.shape[0]
    GW = EBLK // 128

    def body(gp_ref, gq_ref, d0_ref, d1_ref, d2_ref, ea_ref, wr_ref, we_ref,
             w2_ref, b2_ref, w3_ref, t0_ref, t1_ref, t2_ref):
        pre, radial = _edge_pre(gp_ref, gq_ref, d0_ref, d1_ref, d2_ref,
                                ea_ref, wr_ref, we_ref)
        t1m = _silu(pre)
        t2m = _silu(jnp.dot(t1m.astype(jnp.bfloat16),
                            w2_ref[...].astype(jnp.bfloat16),
                            preferred_element_type=jnp.float32) + b2_ref[...])
        phi = jnp.sum(t2m * w3_ref[...], axis=1, keepdims=True)
        scale = 1.0 / ((jnp.sqrt(radial + 1e-8) + 1.0) * NORMV)
        phl = []
        for g in range(EBLK // 128):
            phl.append(phi[g * 128:(g + 1) * 128, :].T)
        phi_lane = jnp.concatenate(phl, axis=0) * scale
        t0_ref[0] = d0_ref[0] * phi_lane
        t1_ref[0] = d1_ref[0] * phi_lane
        t2_ref[0] = d2_ref[0] * phi_lane

    return pl.pallas_call(
        body,
        grid=(E // EBLK,),
        in_specs=[pl.BlockSpec((EBLK, H), lambda i: (i, 0)),
                  pl.BlockSpec((EBLK, H), lambda i: (i, 0)),
                  pl.BlockSpec((1, GW, 128), lambda i: (i, 0, 0)),
                  pl.BlockSpec((1, GW, 128), lambda i: (i, 0, 0)),
                  pl.BlockSpec((1, GW, 128), lambda i: (i, 0, 0)),
                  pl.BlockSpec((1, GW, 128), lambda i: (i, 0, 0)),
                  pl.BlockSpec((1, H), lambda i: (0, 0)),
                  pl.BlockSpec((1, H), lambda i: (0, 0)),
                  pl.BlockSpec((H, H), lambda i: (0, 0)),
                  pl.BlockSpec((1, H), lambda i: (0, 0)),
                  pl.BlockSpec((1, H), lambda i: (0, 0))],
        out_specs=[pl.BlockSpec((1, GW, 128), lambda i: (i, 0, 0))] * 3,
        out_shape=[jax.ShapeDtypeStruct((E // EBLK, GW, 128), jnp.float32)] * 3,
    )(gp, gq, d0, d1, d2, eaf, wr, we, w2, b2, w3t)


def _tc_xout(x, parts):
    N = x.shape[0]

    def body(x_ref, *refs):
        out_ref = refs[-1]
        acc = x_ref[...]
        for r in refs[:-1]:
            acc = acc + r[:, :3]
        out_ref[...] = acc

    na = len(parts)
    return pl.pallas_call(
        body,
        grid=(N // NBLK,),
        in_specs=[pl.BlockSpec((NBLK, 3), lambda i: (i, 0))]
                 + [pl.BlockSpec((NBLK, 16), lambda i: (i, 0))] * na,
        out_specs=pl.BlockSpec((NBLK, 3), lambda i: (i, 0)),
        out_shape=jax.ShapeDtypeStruct((N, 3), jnp.float32),
    )(x, *parts)


def kernel(h, x, edge_index, edge_attr, params):
    N, H = h.shape
    E = edge_index.shape[1]
    row = edge_index[0]
    col = edge_index[1]
    eq = params['eq']

    xpad = jnp.pad(x, ((0, 0), (0, 13)))
    nb = E // EBLK
    gw = EBLK // 128
    d0f, d1f, d2f = _sc_xdiff(xpad, row, col)
    d0 = d0f.reshape(nb, gw, 128)
    d1 = d1f.reshape(nb, gw, 128)
    d2 = d2f.reshape(nb, gw, 128)
    eaf = edge_attr[:, 0].reshape(nb, gw, 128)

    E1 = (E * 3 // 5 // EBLK) * EBLK
    halves = ((0, E1), (E1, E))
    dsl = lambda a, lo, hi: a.reshape(E)[lo:hi].reshape(
        (hi - lo) // EBLK, EBLK // 128, 128)
    hc = H + H // NCORE

    for p in params['gcl']:
        ew1 = p['ew1']
        P, Q = _tc_pq(h, ew1[:H], ew1[H:2 * H], p['eb1'][None, :])
        gs = [_sc_gather_pair(P, Q, row[lo:hi], col[lo:hi])
              for lo, hi in halves]
        efs = [_tc_gcl_edge(gs[j][0], gs[j][1],
                            dsl(d0, *halves[j]), dsl(d1, *halves[j]),
                            dsl(d2, *halves[j]), dsl(eaf, *halves[j]),
                            ew1[2 * H:2 * H + 1], ew1[2 * H + 1:],
                            p['ew2'], p['eb2'][None, :],
                            p['aw'].T, p['ab'][None, :])
               for j in range(2)]
        aggs = [_sc_scatter(efs[j], row[halves[j][0]:halves[j][1]])
                for j in range(2)]
        h = _tc_node(h, aggs, p['nw1'][:H], p['nw1'][H:hc],
                     p['nw1'][hc:], p['nb1'][None, :],
                     p['nw2'], p['nb2'][None, :])

    w1 = eq['w1']
    P, Q = _tc_pq(h, w1[:H], w1[H:2 * H], eq['b1'][None, :])
    gs = [_sc_gather_pair(P, Q, row[lo:hi], col[lo:hi]) for lo, hi in halves]
    ts = [_tc_eq_edge(gs[j][0], gs[j][1],
                      dsl(d0, *halves[j]), dsl(d1, *halves[j]),
                      dsl(d2, *halves[j]), dsl(eaf, *halves[j]),
                      w1[2 * H:2 * H + 1], w1[2 * H + 1:],
                      eq['w2'], eq['b2'][None, :], eq['w3'].T)
          for j in range(2)]
    t0 = jnp.concatenate([ts[0][0].reshape(E1), ts[1][0].reshape(E - E1)])
    t1 = jnp.concatenate([ts[0][1].reshape(E1), ts[1][1].reshape(E - E1)])
    t2 = jnp.concatenate([ts[0][2].reshape(E1), ts[1][2].reshape(E - E1)])
    a0, a1 = _sc_xscatter(t0, t1, t2, row)
    xnew = _tc_xout(x, [a0, a1])
    return h, xnew
